# baseline (speedup 1.0000x reference)
import functools

import jax
import jax.numpy as jnp
from jax import lax
from jax.experimental import pallas as pl
from jax.experimental.pallas import tpu as pltpu
from jax.experimental.pallas import tpu_sc as plsc

N = 10000
E = 320000
D_IN = 128
D_HID = 128
D_OUT = 64

NC = 2
NS = 16
NW = NC * NS
CH = 128
NCH = 80
EPW = NCH * CH
EP = NW * EPW
RPT = 640
NPAD = NS * RPT
NBUF = 2
IKB = 8
PD = 3
DCH = 128
DNCH = EPW // DCH

_mesh = plsc.VectorSubcoreMesh(core_axis_name="c", subcore_axis_name="s")


@functools.partial(
    pl.kernel,
    out_type=jax.ShapeDtypeStruct((NC, NPAD, 16), jnp.float32),
    mesh=_mesh,
    scratch_types=[
        pltpu.VMEM((DNCH, DCH), jnp.int32),
        pltpu.VMEM((DCH, 16), jnp.float32),
        pltpu.VMEM((DCH, 16), jnp.float32),
        pltpu.VMEM_SHARED((NPAD, 16), jnp.float32),
    ],
)
def _sc_deg(dst_hbm, out_hbm, dst_v, ones_v, zero_v, acc_sh):
    c = lax.axis_index("c")
    s = lax.axis_index("s")
    wid = c * NS + s
    row0 = s * RPT

    def fill(i, carry):
        ones_v[i, :] = jnp.ones((16,), jnp.float32)
        zero_v[i, :] = jnp.zeros((16,), jnp.float32)
        return carry

    lax.fori_loop(0, DCH, fill, 0)

    def zero_acc(k, carry):
        pltpu.sync_copy(zero_v, acc_sh.at[pl.ds(row0 + k * DCH, DCH)])
        return carry

    lax.fori_loop(0, RPT // DCH, zero_acc, 0)
    pltpu.sync_copy(dst_hbm.at[wid], dst_v)
    plsc.subcore_barrier()

    def step(j, carry):
        pltpu.sync_copy(ones_v, acc_sh.at[dst_v.at[j]], add=True)
        return carry

    lax.fori_loop(0, DNCH, step, 0)
    plsc.subcore_barrier()
    pltpu.sync_copy(acc_sh.at[pl.ds(row0, RPT)], out_hbm.at[c, pl.ds(row0, RPT)])


@functools.partial(
    pl.kernel,
    out_type=jax.ShapeDtypeStruct((NC, NPAD, D_HID), jnp.float32),
    mesh=_mesh,
    scratch_types=[
        pltpu.VMEM((IKB, CH), jnp.int32),
        pltpu.VMEM((IKB, CH), jnp.int32),
        pltpu.VMEM((NBUF, CH, D_HID), jnp.float32),
        pltpu.VMEM_SHARED((NPAD, D_HID), jnp.float32),
        pltpu.SemaphoreType.DMA,
        pltpu.SemaphoreType.DMA,
        pltpu.SemaphoreType.DMA,
        pltpu.SemaphoreType.DMA,
        pltpu.SemaphoreType.DMA,
        pltpu.SemaphoreType.DMA,
    ],
)
def _sc_agg(zs_hbm, src_hbm, dst_hbm, out_hbm,
            src_rg, dst_rg, rows,
            acc_sh, sg0, sg1, ss0, ss1,
            sem_is, sem_id):
    sem_g = [sg0, sg1]
    sem_s = [ss0, ss1]
    c = lax.axis_index("c")
    s = lax.axis_index("s")
    wid = c * NS + s
    row0 = s * RPT

    def zero_rows(i, carry):
        for j in range(D_HID // 16):
            rows[0, i, pl.ds(j * 16, 16)] = jnp.zeros((16,), jnp.float32)
        return carry

    lax.fori_loop(0, CH, zero_rows, 0)

    def zero_acc(k, carry):
        pltpu.sync_copy(rows.at[0], acc_sh.at[pl.ds(row0 + k * CH, CH)])
        return carry

    lax.fori_loop(0, RPT // CH, zero_acc, 0)
    plsc.subcore_barrier()

    def idx_start(jc):
        jc = jnp.minimum(jc, NCH - 1)
        slot = lax.rem(jc, IKB)
        pltpu.make_async_copy(src_hbm.at[wid, jc], src_rg.at[slot],
                              sem_is).start()
        pltpu.make_async_copy(dst_hbm.at[wid, jc], dst_rg.at[slot],
                              sem_id).start()

    def idx_wait():
        pltpu.make_async_copy(src_hbm.at[wid, 0], src_rg.at[0], sem_is).wait()
        pltpu.make_async_copy(dst_hbm.at[wid, 0], dst_rg.at[0], sem_id).wait()

    def gather(j, b):
        slot = lax.rem(j, IKB)
        return pltpu.make_async_copy(zs_hbm.at[src_rg.at[slot]], rows.at[b],
                                     sem_g[b])

    def scatter_wait(j, b):
        slot = lax.rem(j, IKB)
        pltpu.async_copy(rows.at[b], acc_sh.at[dst_rg.at[slot]],
                         sem_s[b], add=True).wait()

    for q in range(PD + NBUF):
        idx_start(q)
    for q in range(NBUF):
        idx_wait()
    for b in range(NBUF):
        gather(b, b).start()

    def group(t, carry):
        for b in range(NBUF):
            j = t * NBUF + b
            idx_start(j + PD + NBUF)
            idx_wait()
            gather(j, b).wait()
            scatter_wait(j, b)
            gather(j + NBUF, b).start()
        return carry

    lax.fori_loop(0, NCH // NBUF - 1, group, 0)
    for b in range(NBUF):
        j = NCH - NBUF + b
        gather(j, b).wait()
        scatter_wait(j, b)
    for q in range(PD):
        idx_wait()
    plsc.subcore_barrier()
    pltpu.sync_copy(acc_sh.at[pl.ds(row0, RPT)], out_hbm.at[c, pl.ds(row0, RPT)])


def _tc_mm_body(x_ref, w_ref, o_ref):
    o_ref[...] = jnp.dot(x_ref[...], w_ref[...], preferred_element_type=jnp.float32)


_tc_mm = pl.pallas_call(
    _tc_mm_body,
    out_shape=jax.ShapeDtypeStruct((N, D_HID), jnp.float32),
)


def _tc_prep_body(degp_ref, xw_ref, zs_ref, dis_ref):
    deg = degp_ref[0, :N, 0:1] + degp_ref[1, :N, 0:1] + 1.0
    dis = lax.rsqrt(deg)
    dis_ref[...] = dis
    zs_ref[...] = xw_ref[...] * dis


_tc_prep = pl.pallas_call(
    _tc_prep_body,
    out_shape=(
        jax.ShapeDtypeStruct((N, D_HID), jnp.float32),
        jax.ShapeDtypeStruct((N, 1), jnp.float32),
    ),
)


def _tc_mid_body(sp_ref, zs1_ref, dis_ref, b1_ref, zs2_ref):
    dis = dis_ref[...]
    agg = (sp_ref[0, :N, :] + sp_ref[1, :N, :] + zs1_ref[...]) * dis
    h = jnp.maximum(agg + b1_ref[...], 0.0)
    zs2_ref[...] = h * dis


_tc_mid = pl.pallas_call(
    _tc_mid_body,
    out_shape=jax.ShapeDtypeStruct((N, D_HID), jnp.float32),
)


def _tc_out_body(sp_ref, zs2_ref, dis_ref, wmu_ref, bmu_ref, wls_ref, bls_ref,
                 mu_ref, ls_ref):
    agg = (sp_ref[0, :N, :] + sp_ref[1, :N, :] + zs2_ref[...]) * dis_ref[...]
    mu_ref[...] = jnp.dot(agg, wmu_ref[...],
                          preferred_element_type=jnp.float32) + bmu_ref[...]
    ls_ref[...] = jnp.dot(agg, wls_ref[...],
                          preferred_element_type=jnp.float32) + bls_ref[...]


_tc_out = pl.pallas_call(
    _tc_out_body,
    out_shape=(
        jax.ShapeDtypeStruct((N, D_OUT), jnp.float32),
        jax.ShapeDtypeStruct((N, D_OUT), jnp.float32),
    ),
)


def kernel(x, edge_index, W1, b1, W_mu, b_mu, W_ls, b_ls):
    pad_src = jnp.zeros((EP - E,), jnp.int32)
    pad_dst = jnp.full((EP - E,), N, jnp.int32)
    src = jnp.concatenate([edge_index[0], pad_src]).reshape(NW, NCH, CH)
    dst = jnp.concatenate([edge_index[1], pad_dst]).reshape(NW, NCH, CH)
    degp = _sc_deg(dst.reshape(NW, DNCH, DCH))
    xw = _tc_mm(x, W1)
    zs1, dis = _tc_prep(degp, xw)
    s1 = _sc_agg(zs1, src, dst)
    zs2 = _tc_mid(s1, zs1, dis, b1.reshape(1, D_HID))
    s2 = _sc_agg(zs2, src, dst)
    mu, ls = _tc_out(s2, zs2, dis, W_mu, b_mu.reshape(1, D_OUT),
                     W_ls, b_ls.reshape(1, D_OUT))
    return (mu, ls)


# speedup vs baseline: 1.2970x; 1.2970x over previous
import functools

import jax
import jax.numpy as jnp
from jax import lax
from jax.experimental import pallas as pl
from jax.experimental.pallas import tpu as pltpu
from jax.experimental.pallas import tpu_sc as plsc

N = 10000
E = 320000
D_IN = 128
D_HID = 128
D_OUT = 64

NC = 2
NS = 16
NW = NC * NS
CH = 128
NCH = 80
EPW = NCH * CH
EP = NW * EPW
RPT = 640
NPAD = NS * RPT
NBUF = 2
IKB = 8
PD = 3
DCH = 128
DNCH = EPW // DCH

_mesh = plsc.VectorSubcoreMesh(core_axis_name="c", subcore_axis_name="s")


@functools.partial(
    pl.kernel,
    out_type=jax.ShapeDtypeStruct((NC, NPAD, 16), jnp.float32),
    mesh=_mesh,
    scratch_types=[
        pltpu.VMEM((DNCH, DCH), jnp.int32),
        pltpu.VMEM((DCH, 16), jnp.float32),
        pltpu.VMEM((DCH, 16), jnp.float32),
        pltpu.VMEM_SHARED((NPAD, 16), jnp.float32),
    ],
)
def _sc_deg(dst_hbm, out_hbm, dst_v, ones_v, zero_v, acc_sh):
    c = lax.axis_index("c")
    s = lax.axis_index("s")
    wid = c * NS + s
    row0 = s * RPT

    def fill(i, carry):
        ones_v[i, :] = jnp.ones((16,), jnp.float32)
        zero_v[i, :] = jnp.zeros((16,), jnp.float32)
        return carry

    lax.fori_loop(0, DCH, fill, 0)

    def zero_acc(k, carry):
        pltpu.sync_copy(zero_v, acc_sh.at[pl.ds(row0 + k * DCH, DCH)])
        return carry

    lax.fori_loop(0, RPT // DCH, zero_acc, 0)
    pltpu.sync_copy(dst_hbm.at[wid], dst_v)
    plsc.subcore_barrier()

    def step(j, carry):
        pltpu.sync_copy(ones_v, acc_sh.at[dst_v.at[j]], add=True)
        return carry

    lax.fori_loop(0, DNCH, step, 0)
    plsc.subcore_barrier()
    pltpu.sync_copy(acc_sh.at[pl.ds(row0, RPT)], out_hbm.at[c, pl.ds(row0, RPT)])


@functools.partial(
    pl.kernel,
    out_type=jax.ShapeDtypeStruct((NC, NPAD, D_HID), jnp.float32),
    mesh=_mesh,
    scratch_types=[
        pltpu.VMEM((IKB, CH), jnp.int32),
        pltpu.VMEM((IKB, CH), jnp.int32),
        pltpu.VMEM((NBUF, CH, D_HID), jnp.float32),
        pltpu.VMEM_SHARED((NPAD, D_HID), jnp.float32),
        pltpu.SemaphoreType.DMA,
        pltpu.SemaphoreType.DMA,
        pltpu.SemaphoreType.DMA,
        pltpu.SemaphoreType.DMA,
        pltpu.SemaphoreType.DMA,
        pltpu.SemaphoreType.DMA,
    ],
)
def _sc_agg(zs_hbm, src_hbm, dst_hbm, out_hbm,
            src_rg, dst_rg, rows,
            acc_sh, sg0, sg1, ss0, ss1,
            sem_is, sem_id):
    sem_g = [sg0, sg1]
    sem_s = [ss0, ss1]
    c = lax.axis_index("c")
    s = lax.axis_index("s")
    wid = c * NS + s
    row0 = s * RPT

    def zero_rows(i, carry):
        for j in range(D_HID // 16):
            rows[0, i, pl.ds(j * 16, 16)] = jnp.zeros((16,), jnp.float32)
        return carry

    lax.fori_loop(0, CH, zero_rows, 0)

    def zero_acc(k, carry):
        pltpu.sync_copy(rows.at[0], acc_sh.at[pl.ds(row0 + k * CH, CH)])
        return carry

    lax.fori_loop(0, RPT // CH, zero_acc, 0)
    plsc.subcore_barrier()

    def idx_start(jc):
        jc = jnp.minimum(jc, NCH - 1)
        slot = lax.rem(jc, IKB)
        pltpu.make_async_copy(src_hbm.at[wid, jc], src_rg.at[slot],
                              sem_is).start()
        pltpu.make_async_copy(dst_hbm.at[wid, jc], dst_rg.at[slot],
                              sem_id).start()

    def idx_wait():
        pltpu.make_async_copy(src_hbm.at[wid, 0], src_rg.at[0], sem_is).wait()
        pltpu.make_async_copy(dst_hbm.at[wid, 0], dst_rg.at[0], sem_id).wait()

    def gather(j, b):
        slot = lax.rem(j, IKB)
        return pltpu.make_async_copy(zs_hbm.at[src_rg.at[slot]], rows.at[b],
                                     sem_g[b])

    def scatter_wait(j, b):
        slot = lax.rem(j, IKB)
        pltpu.async_copy(rows.at[b], acc_sh.at[dst_rg.at[slot]],
                         sem_s[b], add=True).wait()

    for q in range(PD + NBUF):
        idx_start(q)
    for q in range(NBUF):
        idx_wait()
    for b in range(NBUF):
        gather(b, b).start()

    def group(t, carry):
        for b in range(NBUF):
            j = t * NBUF + b
            idx_start(j + PD + NBUF)
            idx_wait()
            gather(j, b).wait()
            scatter_wait(j, b)
            gather(j + NBUF, b).start()
        return carry

    lax.fori_loop(0, NCH // NBUF - 1, group, 0)
    for b in range(NBUF):
        j = NCH - NBUF + b
        gather(j, b).wait()
        scatter_wait(j, b)
    for q in range(PD):
        idx_wait()
    plsc.subcore_barrier()
    pltpu.sync_copy(acc_sh.at[pl.ds(row0, RPT)], out_hbm.at[c, pl.ds(row0, RPT)])


def _tc_mm_body(x_ref, w_ref, o_ref):
    o_ref[...] = jnp.dot(x_ref[...], w_ref[...], preferred_element_type=jnp.float32)


_tc_mm = pl.pallas_call(
    _tc_mm_body,
    out_shape=jax.ShapeDtypeStruct((N, D_HID), jnp.float32),
)


def _tc_prep_body(degp_ref, xw_ref, zs_ref, dis_ref):
    deg = degp_ref[0, :N, 0:1] + degp_ref[1, :N, 0:1] + 1.0
    dis = lax.rsqrt(deg)
    dis_ref[...] = dis
    zs_ref[...] = xw_ref[...] * dis


_tc_prep = pl.pallas_call(
    _tc_prep_body,
    out_shape=(
        jax.ShapeDtypeStruct((N, D_HID), jnp.float32),
        jax.ShapeDtypeStruct((N, 1), jnp.float32),
    ),
)


def _tc_mid_body(sp_ref, zs1_ref, dis_ref, b1_ref, zs2_ref):
    dis = dis_ref[...]
    agg = (sp_ref[0, :N, :] + sp_ref[1, :N, :] + zs1_ref[...]) * dis
    h = jnp.maximum(agg + b1_ref[...], 0.0)
    zs2_ref[...] = h * dis


_tc_mid = pl.pallas_call(
    _tc_mid_body,
    out_shape=jax.ShapeDtypeStruct((N, D_HID), jnp.float32),
)


def _tc_out_body(sp_ref, zs2_ref, dis_ref, wmu_ref, bmu_ref, wls_ref, bls_ref,
                 mu_ref, ls_ref):
    agg = (sp_ref[0, :N, :] + sp_ref[1, :N, :] + zs2_ref[...]) * dis_ref[...]
    mu_ref[...] = jnp.dot(agg, wmu_ref[...],
                          preferred_element_type=jnp.float32) + bmu_ref[...]
    ls_ref[...] = jnp.dot(agg, wls_ref[...],
                          preferred_element_type=jnp.float32) + bls_ref[...]


_tc_out = pl.pallas_call(
    _tc_out_body,
    out_shape=(
        jax.ShapeDtypeStruct((N, D_OUT), jnp.float32),
        jax.ShapeDtypeStruct((N, D_OUT), jnp.float32),
    ),
)


def kernel(x, edge_index, W1, b1, W_mu, b_mu, W_ls, b_ls):
    pad_src = jnp.zeros((EP - E,), jnp.int32)
    pad_dst = N + jnp.arange(EP - E, dtype=jnp.int32) % (NPAD - N)
    src = (jnp.concatenate([edge_index[0], pad_src])
           .reshape(EPW, NW).T.reshape(NW, NCH, CH))
    dst = (jnp.concatenate([edge_index[1], pad_dst])
           .reshape(EPW, NW).T.reshape(NW, NCH, CH))
    degp = _sc_deg(dst.reshape(NW, DNCH, DCH))
    xw = _tc_mm(x, W1)
    zs1, dis = _tc_prep(degp, xw)
    s1 = _sc_agg(zs1, src, dst)
    zs2 = _tc_mid(s1, zs1, dis, b1.reshape(1, D_HID))
    s2 = _sc_agg(zs2, src, dst)
    mu, ls = _tc_out(s2, zs2, dis, W_mu, b_mu.reshape(1, D_OUT),
                     W_ls, b_ls.reshape(1, D_OUT))
    return (mu, ls)


# speedup vs baseline: 3.5368x; 2.7270x over previous
import functools

import jax
import jax.numpy as jnp
from jax import lax
from jax.experimental import pallas as pl
from jax.experimental.pallas import tpu as pltpu
from jax.experimental.pallas import tpu_sc as plsc

N = 10000
E = 320000
D_IN = 128
D_HID = 128
D_OUT = 64

NC = 2
NS = 16
NW = NC * NS
CH = 128
NCH = 80
EPW = NCH * CH
EP = NW * EPW
RPT = 640
NPAD = NS * RPT
NBUF = 2
IKB = 8
PD = 3
DCH = 128
DNCH = EPW // DCH

_mesh = plsc.VectorSubcoreMesh(core_axis_name="c", subcore_axis_name="s")


@functools.partial(
    pl.kernel,
    out_type=jax.ShapeDtypeStruct((NC, NPAD, 16), jnp.float32),
    mesh=_mesh,
    scratch_types=[
        pltpu.VMEM((DNCH, DCH), jnp.int32),
        pltpu.VMEM((DCH, 16), jnp.float32),
        pltpu.VMEM((DCH, 16), jnp.float32),
        pltpu.VMEM_SHARED((NPAD, 16), jnp.float32),
    ],
)
def _sc_deg(dst_hbm, out_hbm, dst_v, ones_v, zero_v, acc_sh):
    c = lax.axis_index("c")
    s = lax.axis_index("s")
    wid = c * NS + s
    row0 = s * RPT

    def fill(i, carry):
        ones_v[i, :] = jnp.ones((16,), jnp.float32)
        zero_v[i, :] = jnp.zeros((16,), jnp.float32)
        return carry

    lax.fori_loop(0, DCH, fill, 0)

    def zero_acc(k, carry):
        pltpu.sync_copy(zero_v, acc_sh.at[pl.ds(row0 + k * DCH, DCH)])
        return carry

    lax.fori_loop(0, RPT // DCH, zero_acc, 0)
    pltpu.sync_copy(dst_hbm.at[wid], dst_v)
    plsc.subcore_barrier()

    def step(j, carry):
        pltpu.sync_copy(ones_v, acc_sh.at[dst_v.at[j]], add=True)
        return carry

    lax.fori_loop(0, DNCH, step, 0)
    plsc.subcore_barrier()
    pltpu.sync_copy(acc_sh.at[pl.ds(row0, RPT)], out_hbm.at[c, pl.ds(row0, RPT)])


@functools.partial(
    pl.kernel,
    out_type=jax.ShapeDtypeStruct((NC, NPAD, D_HID), jnp.float32),
    mesh=_mesh,
    scratch_types=[
        pltpu.VMEM((IKB, CH), jnp.int32),
        pltpu.VMEM((IKB, CH), jnp.int32),
        pltpu.VMEM((NBUF, CH, D_HID), jnp.float32),
        pltpu.VMEM_SHARED((NPAD, D_HID), jnp.float32),
        pltpu.SemaphoreType.DMA,
        pltpu.SemaphoreType.DMA,
        pltpu.SemaphoreType.DMA,
        pltpu.SemaphoreType.DMA,
        pltpu.SemaphoreType.DMA,
        pltpu.SemaphoreType.DMA,
    ],
)
def _sc_agg(zs_hbm, src_hbm, dst_hbm, out_hbm,
            src_rg, dst_rg, rows,
            acc_sh, sg0, sg1, ss0, ss1,
            sem_is, sem_id):
    sem_g = [sg0, sg1]
    sem_s = [ss0, ss1]
    c = lax.axis_index("c")
    s = lax.axis_index("s")
    wid = c * NS + s
    row0 = s * RPT

    def zero_rows(i, carry):
        for j in range(D_HID // 16):
            rows[0, i, pl.ds(j * 16, 16)] = jnp.zeros((16,), jnp.float32)
        return carry

    lax.fori_loop(0, CH, zero_rows, 0)

    def zero_acc(k, carry):
        pltpu.sync_copy(rows.at[0], acc_sh.at[pl.ds(row0 + k * CH, CH)])
        return carry

    lax.fori_loop(0, RPT // CH, zero_acc, 0)
    plsc.subcore_barrier()

    def idx_start(jc):
        jc = jnp.minimum(jc, NCH - 1)
        slot = lax.rem(jc, IKB)
        pltpu.make_async_copy(src_hbm.at[wid, jc], src_rg.at[slot],
                              sem_is).start()
        pltpu.make_async_copy(dst_hbm.at[wid, jc], dst_rg.at[slot],
                              sem_id).start()

    def idx_wait():
        pltpu.make_async_copy(src_hbm.at[wid, 0], src_rg.at[0], sem_is).wait()
        pltpu.make_async_copy(dst_hbm.at[wid, 0], dst_rg.at[0], sem_id).wait()

    def gather(j, b):
        slot = lax.rem(j, IKB)
        return pltpu.make_async_copy(zs_hbm.at[src_rg.at[slot]], rows.at[b],
                                     sem_g[b])

    def scatter_wait(j, b):
        slot = lax.rem(j, IKB)
        pltpu.async_copy(rows.at[b], acc_sh.at[dst_rg.at[slot]],
                         sem_s[b], add=True).wait()

    for q in range(PD + NBUF):
        idx_start(q)
    for q in range(NBUF):
        idx_wait()
    for b in range(NBUF):
        gather(b, b).start()

    def group(t, carry):
        for b in range(NBUF):
            j = t * NBUF + b
            idx_start(j + PD + NBUF)
            idx_wait()
            gather(j, b).wait()
            scatter_wait(j, b)
            gather(j + NBUF, b).start()
        return carry

    lax.fori_loop(0, NCH // NBUF - 1, group, 0)
    for b in range(NBUF):
        j = NCH - NBUF + b
        gather(j, b).wait()
        scatter_wait(j, b)
    for q in range(PD):
        idx_wait()
    plsc.subcore_barrier()
    pltpu.sync_copy(acc_sh.at[pl.ds(row0, RPT)], out_hbm.at[c, pl.ds(row0, RPT)])


def _tc_mm_body(x_ref, w_ref, o_ref):
    o_ref[...] = jnp.dot(x_ref[...], w_ref[...], preferred_element_type=jnp.float32)


_tc_mm = pl.pallas_call(
    _tc_mm_body,
    out_shape=jax.ShapeDtypeStruct((N, D_HID), jnp.float32),
)


def _tc_prep_body(degp_ref, xw_ref, m_ref, zs_ref, dis_ref):
    deg = degp_ref[0, :N, 0:1] + degp_ref[1, :N, 0:1] + m_ref[...]
    dis = lax.rsqrt(deg)
    dis_ref[...] = dis
    zs_ref[...] = xw_ref[...] * dis


_tc_prep = pl.pallas_call(
    _tc_prep_body,
    out_shape=(
        jax.ShapeDtypeStruct((N, D_HID), jnp.float32),
        jax.ShapeDtypeStruct((N, 1), jnp.float32),
    ),
)


def _tc_mid_body(sp_ref, zs1_ref, dis_ref, m_ref, b1_ref, zs2_ref):
    dis = dis_ref[...]
    agg = (sp_ref[0, :N, :] + sp_ref[1, :N, :] + m_ref[...] * zs1_ref[...]) * dis
    h = jnp.maximum(agg + b1_ref[...], 0.0)
    zs2_ref[...] = h * dis


_tc_mid = pl.pallas_call(
    _tc_mid_body,
    out_shape=jax.ShapeDtypeStruct((N, D_HID), jnp.float32),
)


def _tc_out_body(sp_ref, zs2_ref, dis_ref, m_ref, wmu_ref, bmu_ref, wls_ref,
                 bls_ref, mu_ref, ls_ref):
    agg = (sp_ref[0, :N, :] + sp_ref[1, :N, :]
           + m_ref[...] * zs2_ref[...]) * dis_ref[...]
    mu_ref[...] = jnp.dot(agg, wmu_ref[...],
                          preferred_element_type=jnp.float32) + bmu_ref[...]
    ls_ref[...] = jnp.dot(agg, wls_ref[...],
                          preferred_element_type=jnp.float32) + bls_ref[...]


_tc_out = pl.pallas_call(
    _tc_out_body,
    out_shape=(
        jax.ShapeDtypeStruct((N, D_OUT), jnp.float32),
        jax.ShapeDtypeStruct((N, D_OUT), jnp.float32),
    ),
)


def kernel(x, edge_index, W1, b1, W_mu, b_mu, W_ls, b_ls):
    pad_src = jnp.arange(EP - E, dtype=jnp.int32)
    pad_dst = jnp.arange(EP - E, dtype=jnp.int32)
    src = (jnp.concatenate([edge_index[0], pad_src])
           .reshape(EPW, NW).T.reshape(NW, NCH, CH))
    dst = (jnp.concatenate([edge_index[1], pad_dst])
           .reshape(EPW, NW).T.reshape(NW, NCH, CH))
    m = (jnp.arange(N, dtype=jnp.int32) >= EP - E).astype(jnp.float32)
    m = m.reshape(N, 1)
    degp = _sc_deg(dst.reshape(NW, DNCH, DCH))
    xw = _tc_mm(x, W1)
    zs1, dis = _tc_prep(degp, xw, m)
    s1 = _sc_agg(zs1, src, dst)
    zs2 = _tc_mid(s1, zs1, dis, m, b1.reshape(1, D_HID))
    s2 = _sc_agg(zs2, src, dst)
    mu, ls = _tc_out(s2, zs2, dis, m, W_mu, b_mu.reshape(1, D_OUT),
                     W_ls, b_ls.reshape(1, D_OUT))
    return (mu, ls)
